# Initial kernel scaffold; baseline (speedup 1.0000x reference)
import jax
import jax.numpy as jnp
from jax.experimental import pallas as pl


def kernel(x, edge_index, W0, b0, W1, b1, W2, b2, W3, b3, Wl1, bl1, Wl2, bl2):
    raise NotImplementedError("write your pallas kernel here")


# speedup vs baseline: 5.7350x; 5.7350x over previous
import functools

import jax
import jax.numpy as jnp
from jax import lax
from jax.experimental import pallas as pl
from jax.experimental.pallas import tpu as pltpu
from jax.experimental.pallas import tpu_sc as plsc

_N, _E, _D, _C = 10000, 320000, 128, 4
_NC, _NS = 2, 16
_NW = _NC * _NS
_CH = 80
_NCHT = _E // (_NW * _CH)
_ET = _E // _NW
_ZT = 10
_NPT = _N // _ZT
_GB = 5
_NG = _NCHT // _GB
_ROWB = 1000
_EB = 2000

_HIGH = lax.Precision.HIGHEST


def _mesh():
    return plsc.VectorSubcoreMesh(core_axis_name="c", subcore_axis_name="s")


_FCH = 128
_FNC = 80
_FBLK = 8
_FNB = _FNC // _FBLK


def _sc_segsum(y, srcf, dstf, zeros):
    @functools.partial(
        pl.kernel,
        out_type=jax.ShapeDtypeStruct((_NC, _N, _D), jnp.float32),
        mesh=_mesh(),
        scratch_types=[
            pltpu.VMEM((_FNC, _FCH), jnp.int32),
            pltpu.VMEM((_FBLK, _FCH), jnp.int32),
            pltpu.VMEM((_FCH, _D), jnp.float32),
            pltpu.VMEM((_FCH, _D), jnp.float32),
            pltpu.VMEM_SHARED((_N + 8, _D), jnp.float32),
        ] + [pltpu.SemaphoreType.DMA] * 4,
    )
    def k(y_hbm, srcf_hbm, dstf_hbm, zeros_hbm, out_hbm,
          srcv, dbuf, r0, r1, acc, gs0, gs1, ss0, ss1):
        rows = (r0, r1)
        gsem = (gs0, gs1)
        ssem = (ss0, ss1)
        c = lax.axis_index("c")
        s = lax.axis_index("s")
        tid = c * _NS + s
        pltpu.sync_copy(srcf_hbm.at[tid], srcv)

        @pl.when(s < _ZT)
        def _():
            pltpu.sync_copy(zeros_hbm.at[pl.ds(s * _NPT, _NPT)],
                            acc.at[pl.ds(s * _NPT, _NPT)])

        plsc.subcore_barrier()

        @pl.loop(0, _FNB)
        def _(blk):
            base = blk * _FBLK
            ghs = {}
            shs = {}
            ghs[0] = pltpu.async_copy(y_hbm.at[srcv.at[base]], rows[0],
                                      gsem[0])
            pltpu.sync_copy(dstf_hbm.at[tid, pl.ds(base, _FBLK)], dbuf)
            for i in range(_FBLK):
                b = i & 1
                if i + 1 < _FBLK:
                    nb = (i + 1) & 1
                    if i >= 1:
                        shs[i - 1].wait()
                    ghs[i + 1] = pltpu.async_copy(
                        y_hbm.at[srcv.at[base + i + 1]], rows[nb], gsem[nb])
                ghs[i].wait()
                shs[i] = pltpu.async_copy(rows[b], acc.at[dbuf.at[i]],
                                          ssem[b], add=True)
            shs[_FBLK - 2].wait()
            shs[_FBLK - 1].wait()

        plsc.subcore_barrier()

        @pl.when(s < _ZT)
        def _():
            pltpu.sync_copy(acc.at[pl.ds(s * _NPT, _NPT)],
                            out_hbm.at[c, pl.ds(s * _NPT, _NPT)])

    return k(y, srcf, dstf, zeros)


_NCH2 = _E // (_NS * _CH)
_NG2 = _NCH2 // _GB


def _sc_gather2(hA, hB, src2, dst2):
    @functools.partial(
        pl.kernel,
        out_type=[jax.ShapeDtypeStruct((_E, _D), jnp.float32),
                  jax.ShapeDtypeStruct((_E, _D), jnp.float32)],
        mesh=_mesh(),
        scratch_types=[pltpu.VMEM((_NCH2, _CH), jnp.int32)]
          + [pltpu.VMEM((_CH, _D), jnp.float32)] * _GB
          + [pltpu.SemaphoreType.DMA] * _GB,
    )
    def k(hA_hbm, hB_hbm, src2_hbm, dst2_hbm, gA_hbm, gB_hbm, idxv, *rest):
        rows = rest[:_GB]
        sems = rest[_GB:]
        c = lax.axis_index("c")
        s = lax.axis_index("s")
        ebase = s * (_E // _NS)

        def run(tab_hbm, idx_hbm, out_hbm):
            pltpu.sync_copy(idx_hbm.at[s], idxv)

            @pl.loop(0, _NG2)
            def _(g):
                j0 = g * _GB
                ghs = [pltpu.async_copy(tab_hbm.at[idxv.at[j0 + b]],
                                        rows[b], sems[b])
                       for b in range(_GB)]
                for b in range(_GB):
                    ghs[b].wait()
                    pltpu.sync_copy(
                        rows[b],
                        out_hbm.at[pl.ds(ebase + (j0 + b) * _CH, _CH)])

        @pl.when(c == 0)
        def _():
            run(hA_hbm, src2_hbm, gA_hbm)

        @pl.when(c == 1)
        def _():
            run(hB_hbm, dst2_hbm, gB_hbm)

    return k(hA, hB, src2, dst2)


def _tc_first(x, W0, degp):
    def body(x_ref, w_ref, degp_ref, y_ref, dinv_ref):
        deg = degp_ref[0, :, 0:1] + degp_ref[1, :, 0:1] + 1.0
        dinv = lax.rsqrt(deg)
        dinv_ref[...] = jnp.broadcast_to(dinv, (x_ref.shape[0], _D))
        xw = jnp.dot(x_ref[...], w_ref[...], precision=_HIGH,
                     preferred_element_type=jnp.float32)
        y_ref[...] = dinv * xw

    grid = (_N // _ROWB,)
    return pl.pallas_call(
        body,
        grid=grid,
        in_specs=[
            pl.BlockSpec((_ROWB, _D), lambda i: (i, 0)),
            pl.BlockSpec((_D, _D), lambda i: (0, 0)),
            pl.BlockSpec((_NC, _ROWB, _D), lambda i: (0, i, 0)),
        ],
        out_specs=[
            pl.BlockSpec((_ROWB, _D), lambda i: (i, 0)),
            pl.BlockSpec((_ROWB, _D), lambda i: (i, 0)),
        ],
        out_shape=[jax.ShapeDtypeStruct((_N, _D), jnp.float32),
                   jax.ShapeDtypeStruct((_N, _D), jnp.float32)],
    )(x, W0, degp)


def _tc_mid(y, p, dinv, W, b):
    def body(y_ref, p_ref, dinv_ref, w_ref, b_ref, o_ref):
        z = y_ref[...] + p_ref[0] + p_ref[1]
        h = jnp.maximum(dinv_ref[...] * z + b_ref[...], 0.0)
        o_ref[...] = dinv_ref[...] * jnp.dot(
            h, w_ref[...], precision=_HIGH, preferred_element_type=jnp.float32)

    grid = (_N // _ROWB,)
    return pl.pallas_call(
        body,
        grid=grid,
        in_specs=[
            pl.BlockSpec((_ROWB, _D), lambda i: (i, 0)),
            pl.BlockSpec((_NC, _ROWB, _D), lambda i: (0, i, 0)),
            pl.BlockSpec((_ROWB, _D), lambda i: (i, 0)),
            pl.BlockSpec((_D, _D), lambda i: (0, 0)),
            pl.BlockSpec((1, _D), lambda i: (0, 0)),
        ],
        out_specs=pl.BlockSpec((_ROWB, _D), lambda i: (i, 0)),
        out_shape=jax.ShapeDtypeStruct((_N, _D), jnp.float32),
    )(y, p, dinv, W, b.reshape(1, _D))


def _tc_last(y, p, dinv, b3, WlT, WlB, bl1):
    def body(y_ref, p_ref, dinv_ref, b_ref, wt_ref, wb_ref, bl_ref,
             a_ref, bout_ref):
        z = y_ref[...] + p_ref[0] + p_ref[1]
        h = dinv_ref[...] * z + b_ref[...]
        a_ref[...] = jnp.dot(h, wt_ref[...], precision=_HIGH,
                             preferred_element_type=jnp.float32) + bl_ref[...]
        bout_ref[...] = jnp.dot(h, wb_ref[...], precision=_HIGH,
                                preferred_element_type=jnp.float32)

    grid = (_N // _ROWB,)
    return pl.pallas_call(
        body,
        grid=grid,
        in_specs=[
            pl.BlockSpec((_ROWB, _D), lambda i: (i, 0)),
            pl.BlockSpec((_NC, _ROWB, _D), lambda i: (0, i, 0)),
            pl.BlockSpec((_ROWB, _D), lambda i: (i, 0)),
            pl.BlockSpec((1, _D), lambda i: (0, 0)),
            pl.BlockSpec((_D, _D), lambda i: (0, 0)),
            pl.BlockSpec((_D, _D), lambda i: (0, 0)),
            pl.BlockSpec((1, _D), lambda i: (0, 0)),
        ],
        out_specs=[
            pl.BlockSpec((_ROWB, _D), lambda i: (i, 0)),
            pl.BlockSpec((_ROWB, _D), lambda i: (i, 0)),
        ],
        out_shape=[jax.ShapeDtypeStruct((_N, _D), jnp.float32),
                   jax.ShapeDtypeStruct((_N, _D), jnp.float32)],
    )(y, p, dinv, b3.reshape(1, _D), WlT, WlB, bl1.reshape(1, _D))


def _tc_head(gA, gB, Wl2, bl2):
    def body(ga_ref, gb_ref, w_ref, b_ref, o_ref):
        e = jnp.maximum(ga_ref[...] + gb_ref[...], 0.0)
        logits = jnp.dot(e, w_ref[...], precision=_HIGH,
                         preferred_element_type=jnp.float32) + b_ref[...]
        m = jnp.max(logits, axis=-1, keepdims=True)
        ex = jnp.exp(logits - m)
        o_ref[...] = (logits - m) - jnp.log(jnp.sum(ex, axis=-1, keepdims=True))

    grid = (_E // _EB,)
    return pl.pallas_call(
        body,
        grid=grid,
        in_specs=[
            pl.BlockSpec((_EB, _D), lambda i: (i, 0)),
            pl.BlockSpec((_EB, _D), lambda i: (i, 0)),
            pl.BlockSpec((_D, _C), lambda i: (0, 0)),
            pl.BlockSpec((1, _C), lambda i: (0, 0)),
        ],
        out_specs=pl.BlockSpec((_EB, _C), lambda i: (i, 0)),
        out_shape=jax.ShapeDtypeStruct((_E, _C), jnp.float32),
    )(gA, gB, Wl2, bl2.reshape(1, _C))


def kernel(x, edge_index, W0, b0, W1, b1, W2, b2, W3, b3, Wl1, bl1, Wl2, bl2):
    pad = _FNC * _FCH - _ET
    srcf = jnp.pad(edge_index[0].reshape(_NW, _ET),
                   ((0, 0), (0, pad))).reshape(_NW, _FNC, _FCH)
    dstf = jnp.pad(edge_index[1].reshape(_NW, _ET), ((0, 0), (0, pad)),
                   constant_values=_N).reshape(_NW, _FNC, _FCH)
    zeros = jnp.zeros((_N, _D), jnp.float32)

    degp = _sc_segsum(jnp.ones((_N, _D), jnp.float32), srcf, dstf, zeros)
    y, dinv = _tc_first(x, W0, degp)

    p = _sc_segsum(y, srcf, dstf, zeros)
    y = _tc_mid(y, p, dinv, W1, b0)
    p = _sc_segsum(y, srcf, dstf, zeros)
    y = _tc_mid(y, p, dinv, W2, b1)
    p = _sc_segsum(y, srcf, dstf, zeros)
    y = _tc_mid(y, p, dinv, W3, b2)
    p = _sc_segsum(y, srcf, dstf, zeros)

    hA, hB = _tc_last(y, p, dinv, b3, Wl1[:_D, :], Wl1[_D:, :], bl1)
    src2 = edge_index[0].reshape(_NS, _NCH2, _CH)
    dst2 = edge_index[1].reshape(_NS, _NCH2, _CH)
    gA, gB = _sc_gather2(hA, hB, src2, dst2)
    return _tc_head(gA, gB, Wl2, bl2)


# speedup vs baseline: 6.5122x; 1.1355x over previous
import functools

import jax
import jax.numpy as jnp
from jax import lax
from jax.experimental import pallas as pl
from jax.experimental.pallas import tpu as pltpu
from jax.experimental.pallas import tpu_sc as plsc

_N, _E, _D, _C = 10000, 320000, 128, 4
_NC, _NS = 2, 16
_NW = _NC * _NS
_CH = 80
_NCHT = _E // (_NW * _CH)
_ET = _E // _NW
_ZT = 10
_NPT = _N // _ZT
_GB = 5
_NG = _NCHT // _GB
_ROWB = 1000
_EB = 2000

_HIGH = lax.Precision.HIGHEST


def _mesh():
    return plsc.VectorSubcoreMesh(core_axis_name="c", subcore_axis_name="s")


_FCH = 128
_FNC = 80
_FBLK = 8
_FNB = _FNC // _FBLK


def _sc_degree(dstf, zeros_bf, ones_bf):
    @functools.partial(
        pl.kernel,
        out_type=jax.ShapeDtypeStruct((_NC, _N, _D), jnp.float32),
        mesh=_mesh(),
        scratch_types=[
            pltpu.VMEM((_FNC, _FCH), jnp.int32),
            pltpu.VMEM((_FCH, _D), jnp.float32),
            pltpu.VMEM_SHARED((_N + 8, _D), jnp.float32),
        ] + [pltpu.SemaphoreType.DMA] * 4,
    )
    def k(dstf_hbm, zeros_hbm, ones_hbm, out_hbm, dstv, ones_v, acc, *sems):
        c = lax.axis_index("c")
        s = lax.axis_index("s")
        tid = c * _NS + s
        pltpu.sync_copy(ones_hbm, ones_v)
        pltpu.sync_copy(dstf_hbm.at[tid], dstv)

        @pl.when(s < _ZT)
        def _():
            pltpu.sync_copy(zeros_hbm.at[pl.ds(s * _NPT, _NPT)],
                            acc.at[pl.ds(s * _NPT, _NPT)])

        plsc.subcore_barrier()

        @pl.loop(0, _FNC // 4)
        def _(g):
            hs = [pltpu.async_copy(ones_v, acc.at[dstv.at[g * 4 + b]],
                                   sems[b], add=True)
                  for b in range(4)]
            for h in hs:
                h.wait()

        plsc.subcore_barrier()

        @pl.when(s < _ZT)
        def _():
            pltpu.sync_copy(acc.at[pl.ds(s * _NPT, _NPT)],
                            out_hbm.at[c, pl.ds(s * _NPT, _NPT)])

    return k(dstf, zeros_bf, ones_bf)


def _sc_segsum(y, srcf, dstf, zeros):
    @functools.partial(
        pl.kernel,
        out_type=jax.ShapeDtypeStruct((_NC, _N, _D), jnp.float32),
        mesh=_mesh(),
        scratch_types=[
            pltpu.VMEM((_FNC, _FCH), jnp.int32),
            pltpu.VMEM((_FBLK, _FCH), jnp.int32),
            pltpu.VMEM((_FCH, _D), jnp.float32),
            pltpu.VMEM((_FCH, _D), jnp.float32),
            pltpu.VMEM_SHARED((_N + 8, _D), jnp.float32),
        ] + [pltpu.SemaphoreType.DMA] * 4,
    )
    def k(y_hbm, srcf_hbm, dstf_hbm, zeros_hbm, out_hbm,
          srcv, dbuf, r0, r1, acc, gs0, gs1, ss0, ss1):
        rows = (r0, r1)
        gsem = (gs0, gs1)
        ssem = (ss0, ss1)
        c = lax.axis_index("c")
        s = lax.axis_index("s")
        tid = c * _NS + s
        pltpu.sync_copy(srcf_hbm.at[tid], srcv)

        @pl.when(s < _ZT)
        def _():
            pltpu.sync_copy(zeros_hbm.at[pl.ds(s * _NPT, _NPT)],
                            acc.at[pl.ds(s * _NPT, _NPT)])

        plsc.subcore_barrier()

        @pl.loop(0, _FNB)
        def _(blk):
            base = blk * _FBLK
            ghs = {}
            shs = {}
            ghs[0] = pltpu.async_copy(y_hbm.at[srcv.at[base]], rows[0],
                                      gsem[0])
            pltpu.sync_copy(dstf_hbm.at[tid, pl.ds(base, _FBLK)], dbuf)
            for i in range(_FBLK):
                b = i & 1
                if i + 1 < _FBLK:
                    nb = (i + 1) & 1
                    if i >= 1:
                        shs[i - 1].wait()
                    ghs[i + 1] = pltpu.async_copy(
                        y_hbm.at[srcv.at[base + i + 1]], rows[nb], gsem[nb])
                ghs[i].wait()
                shs[i] = pltpu.async_copy(rows[b], acc.at[dbuf.at[i]],
                                          ssem[b], add=True)
            shs[_FBLK - 2].wait()
            shs[_FBLK - 1].wait()

        plsc.subcore_barrier()

        @pl.when(s < _ZT)
        def _():
            pltpu.sync_copy(acc.at[pl.ds(s * _NPT, _NPT)],
                            out_hbm.at[c, pl.ds(s * _NPT, _NPT)])

    return k(y, srcf, dstf, zeros)


_NCH2 = _E // (_NS * _CH)
_NG2 = _NCH2 // _GB


def _sc_gather2(hA, hB, src2, dst2):
    @functools.partial(
        pl.kernel,
        out_type=[jax.ShapeDtypeStruct((_E, _D), jnp.float32),
                  jax.ShapeDtypeStruct((_E, _D), jnp.float32)],
        mesh=_mesh(),
        scratch_types=[pltpu.VMEM((_NCH2, _CH), jnp.int32)]
          + [pltpu.VMEM((_CH, _D), jnp.float32)] * _GB
          + [pltpu.SemaphoreType.DMA] * _GB,
    )
    def k(hA_hbm, hB_hbm, src2_hbm, dst2_hbm, gA_hbm, gB_hbm, idxv, *rest):
        rows = rest[:_GB]
        sems = rest[_GB:]
        c = lax.axis_index("c")
        s = lax.axis_index("s")
        ebase = s * (_E // _NS)

        def run(tab_hbm, idx_hbm, out_hbm):
            pltpu.sync_copy(idx_hbm.at[s], idxv)

            @pl.loop(0, _NG2)
            def _(g):
                j0 = g * _GB
                ghs = [pltpu.async_copy(tab_hbm.at[idxv.at[j0 + b]],
                                        rows[b], sems[b])
                       for b in range(_GB)]
                for b in range(_GB):
                    ghs[b].wait()
                    pltpu.sync_copy(
                        rows[b],
                        out_hbm.at[pl.ds(ebase + (j0 + b) * _CH, _CH)])

        @pl.when(c == 0)
        def _():
            run(hA_hbm, src2_hbm, gA_hbm)

        @pl.when(c == 1)
        def _():
            run(hB_hbm, dst2_hbm, gB_hbm)

    return k(hA, hB, src2, dst2)


def _tc_first(x, W0, degp):
    def body(x_ref, w_ref, degp_ref, y_ref, dinv_ref):
        deg = degp_ref[0, :, 0:1] + degp_ref[1, :, 0:1] + 1.0
        dinv = lax.rsqrt(deg)
        dinv_ref[...] = jnp.broadcast_to(dinv, (x_ref.shape[0], _D))
        xw = jnp.dot(x_ref[...], w_ref[...], precision=_HIGH,
                     preferred_element_type=jnp.float32)
        y_ref[...] = dinv * xw

    grid = (_N // _ROWB,)
    return pl.pallas_call(
        body,
        grid=grid,
        in_specs=[
            pl.BlockSpec((_ROWB, _D), lambda i: (i, 0)),
            pl.BlockSpec((_D, _D), lambda i: (0, 0)),
            pl.BlockSpec((_NC, _ROWB, _D), lambda i: (0, i, 0)),
        ],
        out_specs=[
            pl.BlockSpec((_ROWB, _D), lambda i: (i, 0)),
            pl.BlockSpec((_ROWB, _D), lambda i: (i, 0)),
        ],
        out_shape=[jax.ShapeDtypeStruct((_N, _D), jnp.float32),
                   jax.ShapeDtypeStruct((_N, _D), jnp.float32)],
    )(x, W0, degp)


def _tc_mid(y, p, dinv, W, b):
    def body(y_ref, p_ref, dinv_ref, w_ref, b_ref, o_ref):
        z = y_ref[...] + p_ref[0] + p_ref[1]
        h = jnp.maximum(dinv_ref[...] * z + b_ref[...], 0.0)
        o_ref[...] = dinv_ref[...] * jnp.dot(
            h, w_ref[...], precision=_HIGH, preferred_element_type=jnp.float32)

    grid = (_N // _ROWB,)
    return pl.pallas_call(
        body,
        grid=grid,
        in_specs=[
            pl.BlockSpec((_ROWB, _D), lambda i: (i, 0)),
            pl.BlockSpec((_NC, _ROWB, _D), lambda i: (0, i, 0)),
            pl.BlockSpec((_ROWB, _D), lambda i: (i, 0)),
            pl.BlockSpec((_D, _D), lambda i: (0, 0)),
            pl.BlockSpec((1, _D), lambda i: (0, 0)),
        ],
        out_specs=pl.BlockSpec((_ROWB, _D), lambda i: (i, 0)),
        out_shape=jax.ShapeDtypeStruct((_N, _D), jnp.float32),
    )(y, p, dinv, W, b.reshape(1, _D))


def _tc_last(y, p, dinv, b3, WlT, WlB, bl1):
    def body(y_ref, p_ref, dinv_ref, b_ref, wt_ref, wb_ref, bl_ref,
             a_ref, bout_ref):
        z = y_ref[...] + p_ref[0] + p_ref[1]
        h = dinv_ref[...] * z + b_ref[...]
        a_ref[...] = jnp.dot(h, wt_ref[...], precision=_HIGH,
                             preferred_element_type=jnp.float32) + bl_ref[...]
        bout_ref[...] = jnp.dot(h, wb_ref[...], precision=_HIGH,
                                preferred_element_type=jnp.float32)

    grid = (_N // _ROWB,)
    return pl.pallas_call(
        body,
        grid=grid,
        in_specs=[
            pl.BlockSpec((_ROWB, _D), lambda i: (i, 0)),
            pl.BlockSpec((_NC, _ROWB, _D), lambda i: (0, i, 0)),
            pl.BlockSpec((_ROWB, _D), lambda i: (i, 0)),
            pl.BlockSpec((1, _D), lambda i: (0, 0)),
            pl.BlockSpec((_D, _D), lambda i: (0, 0)),
            pl.BlockSpec((_D, _D), lambda i: (0, 0)),
            pl.BlockSpec((1, _D), lambda i: (0, 0)),
        ],
        out_specs=[
            pl.BlockSpec((_ROWB, _D), lambda i: (i, 0)),
            pl.BlockSpec((_ROWB, _D), lambda i: (i, 0)),
        ],
        out_shape=[jax.ShapeDtypeStruct((_N, _D), jnp.float32),
                   jax.ShapeDtypeStruct((_N, _D), jnp.float32)],
    )(y, p, dinv, b3.reshape(1, _D), WlT, WlB, bl1.reshape(1, _D))


def _tc_head(gA, gB, Wl2, bl2):
    def body(ga_ref, gb_ref, w_ref, b_ref, o_ref):
        e = jnp.maximum(ga_ref[...] + gb_ref[...], 0.0)
        logits = jnp.dot(e, w_ref[...], precision=_HIGH,
                         preferred_element_type=jnp.float32) + b_ref[...]
        m = jnp.max(logits, axis=-1, keepdims=True)
        ex = jnp.exp(logits - m)
        o_ref[...] = (logits - m) - jnp.log(jnp.sum(ex, axis=-1, keepdims=True))

    grid = (_E // _EB,)
    return pl.pallas_call(
        body,
        grid=grid,
        in_specs=[
            pl.BlockSpec((_EB, _D), lambda i: (i, 0)),
            pl.BlockSpec((_EB, _D), lambda i: (i, 0)),
            pl.BlockSpec((_D, _C), lambda i: (0, 0)),
            pl.BlockSpec((1, _C), lambda i: (0, 0)),
        ],
        out_specs=pl.BlockSpec((_EB, _C), lambda i: (i, 0)),
        out_shape=jax.ShapeDtypeStruct((_E, _C), jnp.float32),
    )(gA, gB, Wl2, bl2.reshape(1, _C))


def kernel(x, edge_index, W0, b0, W1, b1, W2, b2, W3, b3, Wl1, bl1, Wl2, bl2):
    pad = _FNC * _FCH - _ET
    srcf = jnp.pad(edge_index[0].reshape(_NW, _ET),
                   ((0, 0), (0, pad))).reshape(_NW, _FNC, _FCH)
    dstf = jnp.pad(edge_index[1].reshape(_NW, _ET), ((0, 0), (0, pad)),
                   constant_values=_N).reshape(_NW, _FNC, _FCH)
    zeros = jnp.zeros((_N, _D), jnp.float32)

    degp = _sc_degree(dstf, zeros, jnp.ones((_FCH, _D), jnp.float32))
    y, dinv = _tc_first(x, W0, degp)

    p = _sc_segsum(y, srcf, dstf, zeros)
    y = _tc_mid(y, p, dinv, W1, b0)
    p = _sc_segsum(y, srcf, dstf, zeros)
    y = _tc_mid(y, p, dinv, W2, b1)
    p = _sc_segsum(y, srcf, dstf, zeros)
    y = _tc_mid(y, p, dinv, W3, b2)
    p = _sc_segsum(y, srcf, dstf, zeros)

    hA, hB = _tc_last(y, p, dinv, b3, Wl1[:_D, :], Wl1[_D:, :], bl1)
    src2 = edge_index[0].reshape(_NS, _NCH2, _CH)
    dst2 = edge_index[1].reshape(_NS, _NCH2, _CH)
    gA, gB = _sc_gather2(hA, hB, src2, dst2)
    return _tc_head(gA, gB, Wl2, bl2)
